# Initial kernel scaffold; baseline (speedup 1.0000x reference)
import jax
import jax.numpy as jnp
from jax.experimental import pallas as pl


def kernel(train_paris, flag, adj_matrix, r_index, r_val, rel_matrix, att_matrix, ent_matrix, high_adj, ill_ent, ent_semantic_emb, rel_semantic_emb, att_semantic_emb, ent_emb, rel_emb, att_emb, e_att, e_bias, r_att, r_bias, a_att, a_bias, ent_W1, ent_b1, ent_W2, ent_b2, rel_W1, rel_b1, rel_W2, rel_b2, att_W1, att_b1, att_W2, att_b2, g_al, g_ar):
    raise NotImplementedError("write your pallas kernel here")


# speedup vs baseline: 1.0331x; 1.0331x over previous
import functools

import jax
import jax.numpy as jnp
from jax.experimental import pallas as pl
from jax.experimental.pallas import tpu as pltpu

N = 10000
R = 2000
A = 1000
T = 160000
D = 250
DEPTH = 2


_RB = 400
_KB = 512
_H = 512


def _mlp3_body(x1, x2, x3, w11, w12, w13, w21, w22, w23,
               b1s, b2s, out, acc1, acc2, acc3):
    k = pl.program_id(1)
    nk = pl.num_programs(1)

    @pl.when(k == 0)
    def _init():
        acc1[...] = jnp.zeros_like(acc1)
        acc2[...] = jnp.zeros_like(acc2)
        acc3[...] = jnp.zeros_like(acc3)

    acc1[...] += jnp.dot(x1[...], w11[...], preferred_element_type=jnp.float32)
    acc2[...] += jnp.dot(x2[...], w12[...], preferred_element_type=jnp.float32)
    acc3[...] += jnp.dot(x3[...], w13[...], preferred_element_type=jnp.float32)

    @pl.when(k == nk - 1)
    def _epilogue():
        s = None
        for acc, w2, j in ((acc1, w21, 0), (acc2, w22, 1), (acc3, w23, 2)):
            h = jnp.maximum(acc[...] + b1s[j, :][None, :], 0.0)
            r = jnp.maximum(
                jnp.dot(h, w2[...], preferred_element_type=jnp.float32)
                + b2s[j, :][None, :], 0.0) + h
            s = r if s is None else s + r
        out[...] = s


def _sem_mlp3(x1, x2, x3, w1s, w2s, b1s, b2s):
    grid = (N // _RB, 4096 // _KB)
    xspec = pl.BlockSpec((_RB, _KB), lambda i, k: (i, k))
    w1spec = pl.BlockSpec((_KB, _H), lambda i, k: (k, 0))
    w2spec = pl.BlockSpec((_H, _H), lambda i, k: (0, 0))
    bspec = pl.BlockSpec((3, _H), lambda i, k: (0, 0))
    out = pl.pallas_call(
        _mlp3_body,
        grid=grid,
        in_specs=[xspec, xspec, xspec, w1spec, w1spec, w1spec,
                  w2spec, w2spec, w2spec, bspec, bspec],
        out_specs=pl.BlockSpec((_RB, _H), lambda i, k: (i, 0)),
        out_shape=jax.ShapeDtypeStruct((N, _H), jnp.float32),
        scratch_shapes=[pltpu.VMEM((_RB, _H), jnp.float32)] * 3,
        compiler_params=pltpu.CompilerParams(
            dimension_semantics=("parallel", "arbitrary")),
    )(x1, x2, x3, *w1s, *w2s, b1s, b2s)
    return out[:, :500]


def _seg_softmax(v, seg, num):
    m = jax.ops.segment_max(v, seg, num_segments=num)
    m = jnp.where(jnp.isfinite(m), m, 0.0)
    e = jnp.exp(v - m[seg])
    s = jax.ops.segment_sum(e, seg, num_segments=num)
    return e / (s[seg] + 1e-9)


def _avg(adj, emb, num_rows):
    row, col = adj[0], adj[1]
    cnt = jax.ops.segment_sum(jnp.ones(row.shape[0], jnp.float32), row,
                              num_segments=num_rows)
    s = jax.ops.segment_sum(emb[col], row, num_segments=num_rows)
    return s / (cnt[:, None] + 1e-9)


def _encoder(feat, rn, adj, r_val, high_adj, att, bias):
    outs = []
    h = feat
    src, dst = adj[0], adj[1]
    for l in range(DEPTH):
        hs = h[src]
        msg = hs - 2.0 * jnp.sum(hs * rn, axis=-1, keepdims=True) * rn
        score = jax.nn.leaky_relu(jnp.sum(msg * att[l], axis=-1)) + r_val
        alpha = _seg_softmax(score, dst, N)
        h = jnp.tanh(jax.ops.segment_sum(alpha[:, None] * msg, dst,
                                         num_segments=N) + bias[l])
        outs.append(h)
    out = jnp.concatenate(outs, axis=-1)
    hsrc, hdst = high_adj[0], high_adj[1]
    g = jax.ops.segment_sum(out[hsrc], hdst, num_segments=N)
    c = jax.ops.segment_sum(jnp.ones(hsrc.shape[0], jnp.float32), hdst,
                            num_segments=N)
    return out + g / (c[:, None] + 1e-9)


def _diff_gat(x, adj, al, ar):
    src, dst = adj[0], adj[1]
    sl = x @ al
    sr = x @ ar
    score = jax.nn.leaky_relu(sl[src] + sr[dst])
    alpha = _seg_softmax(score, dst, N)
    return jnp.tanh(jax.ops.segment_sum(alpha[:, None] * x[src], dst,
                                        num_segments=N))


def _norm(x):
    return x / (jnp.linalg.norm(x, axis=-1, keepdims=True) + 1e-5)


def _align_loss(emb, pairs, temp=0.1):
    e = _norm(emb)
    l = e[pairs[:, 0]]
    r = e[pairs[:, 1]]
    logits = (l @ r.T) / temp
    lbl = jnp.arange(pairs.shape[0])
    a = jax.nn.log_softmax(logits, axis=-1)[lbl, lbl]
    b = jax.nn.log_softmax(logits.T, axis=-1)[lbl, lbl]
    return -0.5 * (jnp.mean(a) + jnp.mean(b))


def kernel(train_paris, flag, adj_matrix, r_index, r_val, rel_matrix,
           att_matrix, ent_matrix, high_adj, ill_ent, ent_semantic_emb,
           rel_semantic_emb, att_semantic_emb, ent_emb, rel_emb, att_emb,
           e_att, e_bias, r_att, r_bias, a_att, a_bias, ent_W1, ent_b1,
           ent_W2, ent_b2, rel_W1, rel_b1, rel_W2, rel_b2, att_W1, att_b1,
           att_W2, att_b2, g_al, g_ar):
    ent_feature = _avg(ent_matrix, ent_emb, N)
    rel_feature = _avg(rel_matrix, rel_emb, N)
    att_feature = _avg(att_matrix, att_emb, N)

    rel_ids = r_index[1]
    rtab = rel_emb / (jnp.linalg.norm(rel_emb, axis=-1, keepdims=True) + 1e-9)
    rn = rtab[rel_ids]

    e_f = _encoder(ent_feature, rn, adj_matrix, r_val, high_adj, e_att, e_bias)
    r_f = _encoder(rel_feature, rn, adj_matrix, r_val, high_adj, r_att, r_bias)
    a_f = _encoder(att_feature, rn, adj_matrix, r_val, high_adj, a_att, a_bias)
    kg = jnp.concatenate([e_f, r_f, a_f], axis=-1)

    def padw(w):
        return jnp.pad(w, ((0, 0), (0, _H - w.shape[1])))

    def padw2(w):
        return jnp.pad(w, ((0, _H - w.shape[0]), (0, _H - w.shape[1])))

    def padb(b):
        return jnp.pad(b, (0, _H - b.shape[0]))

    sem = _sem_mlp3(
        ent_semantic_emb, rel_semantic_emb, att_semantic_emb,
        [padw(ent_W1), padw(rel_W1), padw(att_W1)],
        [padw2(ent_W2), padw2(rel_W2), padw2(att_W2)],
        jnp.stack([padb(ent_b1), padb(rel_b1), padb(att_b1)]),
        jnp.stack([padb(ent_b2), padb(rel_b2), padb(att_b2)]),
    )

    fo_in = jnp.concatenate([kg, sem], axis=-1)
    fo = _diff_gat(fo_in, ent_matrix, g_al, g_ar)
    out = jnp.concatenate([kg, sem, fo], axis=-1)
    total = (_align_loss(kg, train_paris) + _align_loss(sem, train_paris)
             + _align_loss(out, train_paris))
    return total


# speedup vs baseline: 1.2294x; 1.1901x over previous
import functools

import jax
import jax.numpy as jnp
from jax.experimental import pallas as pl
from jax.experimental.pallas import tpu as pltpu

N = 10000
R = 2000
A = 1000
T = 160000
D = 250
DEPTH = 2


_RB = 400
_KB = 512
_H = 512


def _mlp3_body(x1, x2, x3, w11, w12, w13, w21, w22, w23,
               b1s, b2s, out, acc1, acc2, acc3):
    k = pl.program_id(1)
    nk = pl.num_programs(1)

    @pl.when(k == 0)
    def _init():
        acc1[...] = jnp.zeros_like(acc1)
        acc2[...] = jnp.zeros_like(acc2)
        acc3[...] = jnp.zeros_like(acc3)

    acc1[...] += jnp.dot(x1[...], w11[...], preferred_element_type=jnp.float32)
    acc2[...] += jnp.dot(x2[...], w12[...], preferred_element_type=jnp.float32)
    acc3[...] += jnp.dot(x3[...], w13[...], preferred_element_type=jnp.float32)

    @pl.when(k == nk - 1)
    def _epilogue():
        s = None
        for acc, w2, j in ((acc1, w21, 0), (acc2, w22, 1), (acc3, w23, 2)):
            h = jnp.maximum(acc[...] + b1s[j, :][None, :], 0.0)
            r = jnp.maximum(
                jnp.dot(h, w2[...], preferred_element_type=jnp.float32)
                + b2s[j, :][None, :], 0.0) + h
            s = r if s is None else s + r
        out[...] = s


def _sem_mlp3(x1, x2, x3, w1s, w2s, b1s, b2s):
    grid = (N // _RB, 4096 // _KB)
    xspec = pl.BlockSpec((_RB, _KB), lambda i, k: (i, k))
    w1spec = pl.BlockSpec((_KB, _H), lambda i, k: (k, 0))
    w2spec = pl.BlockSpec((_H, _H), lambda i, k: (0, 0))
    bspec = pl.BlockSpec((3, _H), lambda i, k: (0, 0))
    out = pl.pallas_call(
        _mlp3_body,
        grid=grid,
        in_specs=[xspec, xspec, xspec, w1spec, w1spec, w1spec,
                  w2spec, w2spec, w2spec, bspec, bspec],
        out_specs=pl.BlockSpec((_RB, _H), lambda i, k: (i, 0)),
        out_shape=jax.ShapeDtypeStruct((N, _H), jnp.float32),
        scratch_shapes=[pltpu.VMEM((_RB, _H), jnp.float32)] * 3,
        compiler_params=pltpu.CompilerParams(
            dimension_semantics=("parallel", "arbitrary")),
    )(x1, x2, x3, *w1s, *w2s, b1s, b2s)
    return out[:, :500]


def _seg_softmax(v, seg, num):
    m = jax.ops.segment_max(v, seg, num_segments=num)
    m = jnp.where(jnp.isfinite(m), m, 0.0)
    e = jnp.exp(v - m[seg])
    s = jax.ops.segment_sum(e, seg, num_segments=num)
    return e / (s[seg] + 1e-9)


def _avg3(ent_mat, rel_mat, att_mat, ent_emb, rel_emb, att_emb):
    rows = jnp.concatenate([ent_mat[0], rel_mat[0] + N, att_mat[0] + 2 * N])
    cols = jnp.concatenate([ent_mat[1], rel_mat[1] + N, att_mat[1] + N + R])
    emb = jnp.concatenate([ent_emb, rel_emb, att_emb], axis=0)
    s = jax.ops.segment_sum(emb[cols], rows, num_segments=3 * N)
    cnt = jax.ops.segment_sum(jnp.ones(rows.shape[0], jnp.float32), rows,
                              num_segments=3 * N)
    f = s / (cnt[:, None] + 1e-9)
    return f[:N], f[N:2 * N], f[2 * N:]


def _encoders_fused(feats, rn, adj, r_val, high_adj, atts, biases):
    src, dst = adj[0], adj[1]
    H3 = jnp.stack(feats, axis=1)
    att_l = [jnp.stack([a[l] for a in atts]) for l in range(DEPTH)]
    bias_l = [jnp.stack([b[l] for b in biases]) for l in range(DEPTH)]
    outs = []
    for l in range(DEPTH):
        Hs = H3[src]
        dj = jnp.einsum('tjc,tc->tj', Hs, rn)
        msg = Hs - 2.0 * dj[..., None] * rn[:, None, :]
        sc = (jax.nn.leaky_relu(jnp.einsum('tjc,jc->tj', msg, att_l[l]))
              + r_val[:, None])
        m = jax.ops.segment_max(sc, dst, num_segments=N)
        m = jnp.where(jnp.isfinite(m), m, 0.0)
        e = jnp.exp(sc - m[dst])
        s = jax.ops.segment_sum(e, dst, num_segments=N)
        alpha = e / (s[dst] + 1e-9)
        agg = jax.ops.segment_sum(
            (alpha[..., None] * msg).reshape(T, 750), dst, num_segments=N)
        H3 = jnp.tanh(agg.reshape(N, 3, 250) + bias_l[l][None])
        outs.append(H3)
    OUT = jnp.stack(outs, axis=2).reshape(N, 3 * DEPTH * 250)
    hsrc, hdst = high_adj[0], high_adj[1]
    g = jax.ops.segment_sum(OUT[hsrc], hdst, num_segments=N)
    c = jax.ops.segment_sum(jnp.ones(hsrc.shape[0], jnp.float32), hdst,
                            num_segments=N)
    return OUT + g / (c[:, None] + 1e-9)


def _diff_gat(x, adj, al, ar):
    src, dst = adj[0], adj[1]
    sl = x @ al
    sr = x @ ar
    score = jax.nn.leaky_relu(sl[src] + sr[dst])
    alpha = _seg_softmax(score, dst, N)
    return jnp.tanh(jax.ops.segment_sum(alpha[:, None] * x[src], dst,
                                        num_segments=N))


def _norm(x):
    return x / (jnp.linalg.norm(x, axis=-1, keepdims=True) + 1e-5)


def _align_loss(emb, pairs, temp=0.1):
    e = _norm(emb)
    l = e[pairs[:, 0]]
    r = e[pairs[:, 1]]
    logits = (l @ r.T) / temp
    lbl = jnp.arange(pairs.shape[0])
    a = jax.nn.log_softmax(logits, axis=-1)[lbl, lbl]
    b = jax.nn.log_softmax(logits.T, axis=-1)[lbl, lbl]
    return -0.5 * (jnp.mean(a) + jnp.mean(b))


def kernel(train_paris, flag, adj_matrix, r_index, r_val, rel_matrix,
           att_matrix, ent_matrix, high_adj, ill_ent, ent_semantic_emb,
           rel_semantic_emb, att_semantic_emb, ent_emb, rel_emb, att_emb,
           e_att, e_bias, r_att, r_bias, a_att, a_bias, ent_W1, ent_b1,
           ent_W2, ent_b2, rel_W1, rel_b1, rel_W2, rel_b2, att_W1, att_b1,
           att_W2, att_b2, g_al, g_ar):
    ent_feature, rel_feature, att_feature = _avg3(
        ent_matrix, rel_matrix, att_matrix, ent_emb, rel_emb, att_emb)

    rel_ids = r_index[1]
    rtab = rel_emb / (jnp.linalg.norm(rel_emb, axis=-1, keepdims=True) + 1e-9)
    rn = rtab[rel_ids]

    kg = _encoders_fused(
        [ent_feature, rel_feature, att_feature], rn, adj_matrix, r_val,
        high_adj, [e_att, r_att, a_att], [e_bias, r_bias, a_bias])

    def padw(w):
        return jnp.pad(w, ((0, 0), (0, _H - w.shape[1])))

    def padw2(w):
        return jnp.pad(w, ((0, _H - w.shape[0]), (0, _H - w.shape[1])))

    def padb(b):
        return jnp.pad(b, (0, _H - b.shape[0]))

    sem = _sem_mlp3(
        ent_semantic_emb, rel_semantic_emb, att_semantic_emb,
        [padw(ent_W1), padw(rel_W1), padw(att_W1)],
        [padw2(ent_W2), padw2(rel_W2), padw2(att_W2)],
        jnp.stack([padb(ent_b1), padb(rel_b1), padb(att_b1)]),
        jnp.stack([padb(ent_b2), padb(rel_b2), padb(att_b2)]),
    )

    fo_in = jnp.concatenate([kg, sem], axis=-1)
    fo = _diff_gat(fo_in, ent_matrix, g_al, g_ar)
    out = jnp.concatenate([kg, sem, fo], axis=-1)
    total = (_align_loss(kg, train_paris) + _align_loss(sem, train_paris)
             + _align_loss(out, train_paris))
    return total


# speedup vs baseline: 1.3071x; 1.0631x over previous
import functools

import jax
import jax.numpy as jnp
from jax import lax
from jax.experimental import pallas as pl
from jax.experimental.pallas import tpu as pltpu
from jax.experimental.pallas import tpu_sc as plsc

N = 10000
R = 2000
A = 1000
T = 160000
D = 250
DEPTH = 2


_RB = 400
_KB = 512
_H = 512


def _mlp3_body(x1, x2, x3, w11, w12, w13, w21, w22, w23,
               b1s, b2s, out, acc1, acc2, acc3):
    k = pl.program_id(1)
    nk = pl.num_programs(1)

    @pl.when(k == 0)
    def _init():
        acc1[...] = jnp.zeros_like(acc1)
        acc2[...] = jnp.zeros_like(acc2)
        acc3[...] = jnp.zeros_like(acc3)

    acc1[...] += jnp.dot(x1[...], w11[...], preferred_element_type=jnp.float32)
    acc2[...] += jnp.dot(x2[...], w12[...], preferred_element_type=jnp.float32)
    acc3[...] += jnp.dot(x3[...], w13[...], preferred_element_type=jnp.float32)

    @pl.when(k == nk - 1)
    def _epilogue():
        s = None
        for acc, w2, j in ((acc1, w21, 0), (acc2, w22, 1), (acc3, w23, 2)):
            h = jnp.maximum(acc[...] + b1s[j, :][None, :], 0.0)
            r = jnp.maximum(
                jnp.dot(h, w2[...], preferred_element_type=jnp.float32)
                + b2s[j, :][None, :], 0.0) + h
            s = r if s is None else s + r
        out[...] = s


def _sem_mlp3(x1, x2, x3, w1s, w2s, b1s, b2s):
    grid = (N // _RB, 4096 // _KB)
    xspec = pl.BlockSpec((_RB, _KB), lambda i, k: (i, k))
    w1spec = pl.BlockSpec((_KB, _H), lambda i, k: (k, 0))
    w2spec = pl.BlockSpec((_H, _H), lambda i, k: (0, 0))
    bspec = pl.BlockSpec((3, _H), lambda i, k: (0, 0))
    out = pl.pallas_call(
        _mlp3_body,
        grid=grid,
        in_specs=[xspec, xspec, xspec, w1spec, w1spec, w1spec,
                  w2spec, w2spec, w2spec, bspec, bspec],
        out_specs=pl.BlockSpec((_RB, _H), lambda i, k: (i, 0)),
        out_shape=jax.ShapeDtypeStruct((N, _H), jnp.float32),
        scratch_shapes=[pltpu.VMEM((_RB, _H), jnp.float32)] * 3,
        compiler_params=pltpu.CompilerParams(
            dimension_semantics=("parallel", "arbitrary")),
    )(x1, x2, x3, *w1s, *w2s, b1s, b2s)
    return out[:, :500]


_NPAD = 10240
_EC = 512
_CB = 64


def _sc_seg_gather_sum(tableb, src2d, dst2d, nblk, vrows, with_counts=True):
    TP = src2d.shape[0] * 128
    chunks_per_tile = TP // (16 * _EC)
    obl = nblk + (1 if with_counts else 0)
    nb_half = nblk // 2
    assert nblk % 2 == 0

    mesh = plsc.VectorSubcoreMesh(core_axis_name="c", subcore_axis_name="s")

    @functools.partial(
        pl.kernel,
        out_type=jax.ShapeDtypeStruct((obl * _NPAD, _CB), jnp.float32),
        mesh=mesh,
        scratch_types=[
            pltpu.VMEM((4, 128), jnp.int32),
            pltpu.VMEM((4, 128), jnp.int32),
            pltpu.VMEM((_EC, _CB), jnp.float32),
            pltpu.VMEM((160, _CB), jnp.float32),
            pltpu.VMEM_SHARED((_NPAD, _CB), jnp.float32),
            pltpu.SemaphoreType.DMA,
        ],
        compiler_params=pltpu.CompilerParams(use_tc_tiling_on_sc=False),
    )
    def k(tab_hbm, src_hbm, dst_hbm, g_hbm, idx_a, idx_d, rows,
          zbuf, acc, sem):
        cid = lax.axis_index("c")
        sid = lax.axis_index("s")

        def zrow(i, _):
            for j in range(_CB // 16):
                zbuf[i, pl.ds(j * 16, 16)] = jnp.zeros((16,), jnp.float32)
            return 0
        lax.fori_loop(0, 160, zrow, 0)

        def zero_acc():
            for r in range(4):
                pltpu.sync_copy(zbuf,
                                acc.at[pl.ds(sid * 640 + r * 160, 160), :])

        def edge_pass(blk, gather):
            zero_acc()
            plsc.subcore_barrier()

            def per_chunk(kk, _):
                rbase = (sid * chunks_per_tile + kk) * (_EC // 128)
                pltpu.sync_copy(dst_hbm.at[pl.ds(rbase, 4), :], idx_d)
                if gather:
                    pltpu.sync_copy(src_hbm.at[pl.ds(rbase, 4), :], idx_a)

                    def adj(i, _):
                        for j in range(4):
                            idx_a[j, pl.ds(i * 16, 16)] = (
                                idx_a[j, pl.ds(i * 16, 16)] + blk * vrows)
                        return 0
                    lax.fori_loop(0, 8, adj, 0)
                    for j in range(4):
                        pltpu.async_copy(
                            tab_hbm.at[idx_a.at[j]],
                            rows.at[pl.ds(j * 128, 128), :], sem).wait()
                for j in range(4):
                    pltpu.sync_copy(rows.at[pl.ds(j * 128, 128), :],
                                    acc.at[idx_d.at[j]], add=True)
                return 0
            lax.fori_loop(0, chunks_per_tile, per_chunk, 0)
            plsc.subcore_barrier()
            pltpu.sync_copy(
                acc.at[pl.ds(sid * 640, 640), :],
                g_hbm.at[pl.ds(blk * _NPAD + sid * 640, 640), :])
            plsc.subcore_barrier()

        for b in range(nb_half):
            edge_pass(2 * b + cid, True)

        if with_counts:
            def basis(i, _):
                rows[i, pl.ds(0, 16)] = jnp.where(
                    lax.iota(jnp.int32, 16) == 0, 1.0, 0.0)
                for j in range(1, _CB // 16):
                    rows[i, pl.ds(j * 16, 16)] = jnp.zeros((16,),
                                                           jnp.float32)
                return 0

            @pl.when(cid == 0)
            def _():
                lax.fori_loop(0, _EC, basis, 0)
                edge_pass(nblk, False)

    return k(tableb, src2d, dst2d)


def _pad_edges(src, dst, dump, multiple=16 * _EC):
    t = src.shape[0]
    tp = ((t + multiple - 1) // multiple) * multiple
    pad = tp - t
    src = jnp.concatenate([src, jnp.zeros((pad,), jnp.int32)])
    dst = jnp.concatenate([dst, jnp.full((pad,), dump, jnp.int32)])
    return src, dst


def _seg_softmax(v, seg, num):
    m = jax.ops.segment_max(v, seg, num_segments=num)
    m = jnp.where(jnp.isfinite(m), m, 0.0)
    e = jnp.exp(v - m[seg])
    s = jax.ops.segment_sum(e, seg, num_segments=num)
    return e / (s[seg] + 1e-9)


def _avg3(ent_mat, rel_mat, att_mat, ent_emb, rel_emb, att_emb):
    rows = jnp.concatenate([ent_mat[0], rel_mat[0] + N, att_mat[0] + 2 * N])
    cols = jnp.concatenate([ent_mat[1], rel_mat[1] + N, att_mat[1] + N + R])
    emb = jnp.concatenate([ent_emb, rel_emb, att_emb], axis=0)
    s = jax.ops.segment_sum(emb[cols], rows, num_segments=3 * N)
    cnt = jax.ops.segment_sum(jnp.ones(rows.shape[0], jnp.float32), rows,
                              num_segments=3 * N)
    f = s / (cnt[:, None] + 1e-9)
    return f[:N], f[N:2 * N], f[2 * N:]


def _encoders_fused(feats, rn, adj, r_val, high_adj, atts, biases):
    src, dst = adj[0], adj[1]
    H3 = jnp.stack(feats, axis=1)
    att_l = [jnp.stack([a[l] for a in atts]) for l in range(DEPTH)]
    bias_l = [jnp.stack([b[l] for b in biases]) for l in range(DEPTH)]
    outs = []
    for l in range(DEPTH):
        Hs = H3[src]
        dj = jnp.einsum('tjc,tc->tj', Hs, rn)
        msg = Hs - 2.0 * dj[..., None] * rn[:, None, :]
        sc = (jax.nn.leaky_relu(jnp.einsum('tjc,jc->tj', msg, att_l[l]))
              + r_val[:, None])
        m = jax.ops.segment_max(sc, dst, num_segments=N)
        m = jnp.where(jnp.isfinite(m), m, 0.0)
        e = jnp.exp(sc - m[dst])
        s = jax.ops.segment_sum(e, dst, num_segments=N)
        alpha = e / (s[dst] + 1e-9)
        agg = jax.ops.segment_sum(
            (alpha[..., None] * msg).reshape(T, 750), dst, num_segments=N)
        H3 = jnp.tanh(agg.reshape(N, 3, 250) + bias_l[l][None])
        outs.append(H3)
    OUT = jnp.stack(outs, axis=2).reshape(N, 3 * DEPTH * 250)
    nbl = 24
    OUTb = jnp.pad(OUT, ((0, 0), (0, nbl * _CB - 1500)))
    OUTb = OUTb.reshape(N, nbl, _CB).transpose(1, 0, 2).reshape(nbl * N, _CB)
    hsrc, hdst = _pad_edges(high_adj[0], high_adj[1], N)
    r = _sc_seg_gather_sum(OUTb, hsrc.reshape(-1, 128), hdst.reshape(-1, 128),
                           nbl, N).reshape(nbl + 1, _NPAD, _CB)
    g = r[:nbl, :N].transpose(1, 0, 2).reshape(N, nbl * _CB)[:, :1500]
    c = r[nbl, :N, 0]
    return OUT + g / (c[:, None] + 1e-9)


def _diff_gat(x, adj, al, ar):
    src, dst = adj[0], adj[1]
    sl = x @ al
    sr = x @ ar
    score = jax.nn.leaky_relu(sl[src] + sr[dst])
    alpha = _seg_softmax(score, dst, N)
    return jnp.tanh(jax.ops.segment_sum(alpha[:, None] * x[src], dst,
                                        num_segments=N))


def _norm(x):
    return x / (jnp.linalg.norm(x, axis=-1, keepdims=True) + 1e-5)


def _align_loss(emb, pairs, temp=0.1):
    e = _norm(emb)
    l = e[pairs[:, 0]]
    r = e[pairs[:, 1]]
    logits = (l @ r.T) / temp
    lbl = jnp.arange(pairs.shape[0])
    a = jax.nn.log_softmax(logits, axis=-1)[lbl, lbl]
    b = jax.nn.log_softmax(logits.T, axis=-1)[lbl, lbl]
    return -0.5 * (jnp.mean(a) + jnp.mean(b))


def kernel(train_paris, flag, adj_matrix, r_index, r_val, rel_matrix,
           att_matrix, ent_matrix, high_adj, ill_ent, ent_semantic_emb,
           rel_semantic_emb, att_semantic_emb, ent_emb, rel_emb, att_emb,
           e_att, e_bias, r_att, r_bias, a_att, a_bias, ent_W1, ent_b1,
           ent_W2, ent_b2, rel_W1, rel_b1, rel_W2, rel_b2, att_W1, att_b1,
           att_W2, att_b2, g_al, g_ar):
    ent_feature, rel_feature, att_feature = _avg3(
        ent_matrix, rel_matrix, att_matrix, ent_emb, rel_emb, att_emb)

    rel_ids = r_index[1]
    rtab = rel_emb / (jnp.linalg.norm(rel_emb, axis=-1, keepdims=True) + 1e-9)
    rn = rtab[rel_ids]

    kg = _encoders_fused(
        [ent_feature, rel_feature, att_feature], rn, adj_matrix, r_val,
        high_adj, [e_att, r_att, a_att], [e_bias, r_bias, a_bias])

    def padw(w):
        return jnp.pad(w, ((0, 0), (0, _H - w.shape[1])))

    def padw2(w):
        return jnp.pad(w, ((0, _H - w.shape[0]), (0, _H - w.shape[1])))

    def padb(b):
        return jnp.pad(b, (0, _H - b.shape[0]))

    sem = _sem_mlp3(
        ent_semantic_emb, rel_semantic_emb, att_semantic_emb,
        [padw(ent_W1), padw(rel_W1), padw(att_W1)],
        [padw2(ent_W2), padw2(rel_W2), padw2(att_W2)],
        jnp.stack([padb(ent_b1), padb(rel_b1), padb(att_b1)]),
        jnp.stack([padb(ent_b2), padb(rel_b2), padb(att_b2)]),
    )

    fo_in = jnp.concatenate([kg, sem], axis=-1)
    fo = _diff_gat(fo_in, ent_matrix, g_al, g_ar)
    out = jnp.concatenate([kg, sem, fo], axis=-1)
    total = (_align_loss(kg, train_paris) + _align_loss(sem, train_paris)
             + _align_loss(out, train_paris))
    return total


# speedup vs baseline: 1.3966x; 1.0685x over previous
import functools

import jax
import jax.numpy as jnp
from jax import lax
from jax.experimental import pallas as pl
from jax.experimental.pallas import tpu as pltpu
from jax.experimental.pallas import tpu_sc as plsc

N = 10000
R = 2000
A = 1000
T = 160000
D = 250
DEPTH = 2


_RB = 400
_KB = 512
_H = 512


def _mlp3_body(x1, x2, x3, w11, w12, w13, w21, w22, w23,
               b1s, b2s, out, acc1, acc2, acc3):
    k = pl.program_id(1)
    nk = pl.num_programs(1)

    @pl.when(k == 0)
    def _init():
        acc1[...] = jnp.zeros_like(acc1)
        acc2[...] = jnp.zeros_like(acc2)
        acc3[...] = jnp.zeros_like(acc3)

    acc1[...] += jnp.dot(x1[...], w11[...], preferred_element_type=jnp.float32)
    acc2[...] += jnp.dot(x2[...], w12[...], preferred_element_type=jnp.float32)
    acc3[...] += jnp.dot(x3[...], w13[...], preferred_element_type=jnp.float32)

    @pl.when(k == nk - 1)
    def _epilogue():
        s = None
        for acc, w2, j in ((acc1, w21, 0), (acc2, w22, 1), (acc3, w23, 2)):
            h = jnp.maximum(acc[...] + b1s[j, :][None, :], 0.0)
            r = jnp.maximum(
                jnp.dot(h, w2[...], preferred_element_type=jnp.float32)
                + b2s[j, :][None, :], 0.0) + h
            s = r if s is None else s + r
        out[...] = s


def _sem_mlp3(x1, x2, x3, w1s, w2s, b1s, b2s):
    grid = (N // _RB, 4096 // _KB)
    xspec = pl.BlockSpec((_RB, _KB), lambda i, k: (i, k))
    w1spec = pl.BlockSpec((_KB, _H), lambda i, k: (k, 0))
    w2spec = pl.BlockSpec((_H, _H), lambda i, k: (0, 0))
    bspec = pl.BlockSpec((3, _H), lambda i, k: (0, 0))
    out = pl.pallas_call(
        _mlp3_body,
        grid=grid,
        in_specs=[xspec, xspec, xspec, w1spec, w1spec, w1spec,
                  w2spec, w2spec, w2spec, bspec, bspec],
        out_specs=pl.BlockSpec((_RB, _H), lambda i, k: (i, 0)),
        out_shape=jax.ShapeDtypeStruct((N, _H), jnp.float32),
        scratch_shapes=[pltpu.VMEM((_RB, _H), jnp.float32)] * 3,
        compiler_params=pltpu.CompilerParams(
            dimension_semantics=("parallel", "arbitrary")),
    )(x1, x2, x3, *w1s, *w2s, b1s, b2s)
    return out[:, :500]


_NPAD = 10240
_EC = 512
_CB = 64


def _sc_multiblock(tabb, src2d, dst2d, entries, nout):
    mesh = plsc.VectorSubcoreMesh(core_axis_name="c", subcore_axis_name="s")

    @functools.partial(
        pl.kernel,
        out_type=jax.ShapeDtypeStruct((nout * _NPAD, _CB), jnp.float32),
        mesh=mesh,
        scratch_types=[
            pltpu.VMEM((4, 128), jnp.int32),
            pltpu.VMEM((4, 128), jnp.int32),
            pltpu.VMEM((_EC, _CB), jnp.float32),
            pltpu.VMEM((160, _CB), jnp.float32),
            pltpu.VMEM_SHARED((_NPAD, _CB), jnp.float32),
            pltpu.SemaphoreType.DMA,
        ],
        compiler_params=pltpu.CompilerParams(use_tc_tiling_on_sc=False),
    )
    def k(tab_hbm, src_hbm, dst_hbm, g_hbm, idx_a, idx_d, rows,
          zbuf, acc, sem):
        cid = lax.axis_index("c")
        sid = lax.axis_index("s")

        def zrow(i, _):
            for j in range(_CB // 16):
                zbuf[i, pl.ds(j * 16, 16)] = jnp.zeros((16,), jnp.float32)
            return 0
        lax.fori_loop(0, 160, zrow, 0)

        def basis(i, _):
            rows[i, pl.ds(0, 16)] = jnp.where(
                lax.iota(jnp.int32, 16) == 0, 1.0, 0.0)
            for j in range(1, _CB // 16):
                rows[i, pl.ds(j * 16, 16)] = jnp.zeros((16,), jnp.float32)
            return 0

        def edge_pass(tab_base, erow_base, cpt, out_blk, gather):
            if not gather:
                lax.fori_loop(0, _EC, basis, 0)
            for r in range(4):
                pltpu.sync_copy(zbuf,
                                acc.at[pl.ds(sid * 640 + r * 160, 160), :])
            plsc.subcore_barrier()

            def per_chunk(kk, _):
                rbase = erow_base + (sid * cpt + kk) * (_EC // 128)
                pltpu.sync_copy(dst_hbm.at[pl.ds(rbase, 4), :], idx_d)
                if gather:
                    pltpu.sync_copy(src_hbm.at[pl.ds(rbase, 4), :], idx_a)
                    if tab_base:
                        def adj(i, _):
                            for j in range(4):
                                idx_a[j, pl.ds(i * 16, 16)] = (
                                    idx_a[j, pl.ds(i * 16, 16)] + tab_base)
                            return 0
                        lax.fori_loop(0, 8, adj, 0)
                    for j in range(4):
                        pltpu.async_copy(
                            tab_hbm.at[idx_a.at[j]],
                            rows.at[pl.ds(j * 128, 128), :], sem).wait()
                for j in range(4):
                    pltpu.sync_copy(rows.at[pl.ds(j * 128, 128), :],
                                    acc.at[idx_d.at[j]], add=True)
                return 0
            lax.fori_loop(0, cpt, per_chunk, 0)
            plsc.subcore_barrier()
            pltpu.sync_copy(
                acc.at[pl.ds(sid * 640, 640), :],
                g_hbm.at[pl.ds(out_blk * _NPAD + sid * 640, 640), :])
            plsc.subcore_barrier()

        for i, (tb, eb, cpt, ob, ga) in enumerate(entries):
            pl.when(cid == i % 2)(
                functools.partial(edge_pass, tb, eb, cpt, ob, ga))

    return k(tabb, src2d, dst2d)


def _pad_edges(src, dst, dump, multiple=16 * _EC):
    t = src.shape[0]
    tp = ((t + multiple - 1) // multiple) * multiple
    pad = tp - t
    src = jnp.concatenate([src, jnp.zeros((pad,), jnp.int32)])
    dst = jnp.concatenate([dst, jnp.full((pad,), dump, jnp.int32)])
    return src, dst


def _seg_softmax(v, seg, num):
    m = jax.ops.segment_max(v, seg, num_segments=num)
    m = jnp.where(jnp.isfinite(m), m, 0.0)
    e = jnp.exp(v - m[seg])
    s = jax.ops.segment_sum(e, seg, num_segments=num)
    return e / (s[seg] + 1e-9)


def _avg3(ent_mat, rel_mat, att_mat, ent_emb, rel_emb, att_emb):
    def padt(t):
        return jnp.pad(t, ((0, _NPAD - t.shape[0]), (0, 256 - t.shape[1])))

    tabp = jnp.stack([padt(ent_emb), padt(rel_emb), padt(att_emb)])
    tabb = tabp.reshape(3, _NPAD, 4, _CB).transpose(0, 2, 1, 3)
    tabb = tabb.reshape(12 * _NPAD, _CB)

    srcs, dsts, ebase, cpts = [], [], [], []
    rb = 0
    for mat in (ent_mat, rel_mat, att_mat):
        c, d = _pad_edges(mat[1], mat[0], N)
        srcs.append(c)
        dsts.append(d)
        ebase.append(rb)
        rb += c.shape[0] // 128
        cpts.append(c.shape[0] // (16 * _EC))
    src2d = jnp.concatenate(srcs).reshape(-1, 128)
    dst2d = jnp.concatenate(dsts).reshape(-1, 128)

    entries = [((j * 4 + q) * _NPAD, ebase[j], cpts[j], j * 4 + q, True)
               for j in range(3) for q in range(4)]
    entries += [(0, ebase[j], cpts[j], 12 + j, False) for j in range(3)]
    r = _sc_multiblock(tabb, src2d, dst2d, entries, 15)
    r = r.reshape(15, _NPAD, _CB)
    s = r[:12].reshape(3, 4, _NPAD, _CB).transpose(2, 0, 1, 3)
    s = s.reshape(_NPAD, 3, 256)[:N, :, :250]
    cnt = r[12:15, :N, 0]
    return s / (cnt.T[:, :, None] + 1e-9)


def _encoders_fused(H3, rn, adj, r_val, high_adj, atts, biases):
    src, dst = adj[0], adj[1]
    att_l = [jnp.stack([a[l] for a in atts]) for l in range(DEPTH)]
    bias_l = [jnp.stack([b[l] for b in biases]) for l in range(DEPTH)]
    outs = []
    for l in range(DEPTH):
        Hs = H3[src]
        dj = jnp.einsum('tjc,tc->tj', Hs, rn)
        msg = Hs - 2.0 * dj[..., None] * rn[:, None, :]
        sc = (jax.nn.leaky_relu(jnp.einsum('tjc,jc->tj', msg, att_l[l]))
              + r_val[:, None])
        m = jax.ops.segment_max(sc, dst, num_segments=N)
        m = jnp.where(jnp.isfinite(m), m, 0.0)
        e = jnp.exp(sc - m[dst])
        s = jax.ops.segment_sum(e, dst, num_segments=N)
        alpha = e / (s[dst] + 1e-9)
        agg = jax.ops.segment_sum(
            (alpha[..., None] * msg).reshape(T, 750), dst, num_segments=N)
        H3 = jnp.tanh(agg.reshape(N, 3, 250) + bias_l[l][None])
        outs.append(H3)
    OUT = jnp.stack(outs, axis=2).reshape(N, 3 * DEPTH * 250)
    nbl = 24
    OUTb = jnp.pad(OUT, ((0, 0), (0, nbl * _CB - 1500)))
    OUTb = OUTb.reshape(N, nbl, _CB).transpose(1, 0, 2).reshape(nbl * N, _CB)
    hsrc, hdst = _pad_edges(high_adj[0], high_adj[1], N)
    cpt = hsrc.shape[0] // (16 * _EC)
    entries = [(b * N, 0, cpt, b, True) for b in range(nbl)]
    entries += [(0, 0, cpt, nbl, False)]
    r = _sc_multiblock(OUTb, hsrc.reshape(-1, 128), hdst.reshape(-1, 128),
                       entries, nbl + 1).reshape(nbl + 1, _NPAD, _CB)
    g = r[:nbl, :N].transpose(1, 0, 2).reshape(N, nbl * _CB)[:, :1500]
    c = r[nbl, :N, 0]
    return OUT + g / (c[:, None] + 1e-9)


def _diff_gat(x, adj, al, ar):
    src, dst = adj[0], adj[1]
    sl = x @ al
    sr = x @ ar
    score = jax.nn.leaky_relu(sl[src] + sr[dst])
    alpha = _seg_softmax(score, dst, N)
    return jnp.tanh(jax.ops.segment_sum(alpha[:, None] * x[src], dst,
                                        num_segments=N))


def _norm(x):
    return x / (jnp.linalg.norm(x, axis=-1, keepdims=True) + 1e-5)


def _align_loss(emb, pairs, temp=0.1):
    e = _norm(emb)
    l = e[pairs[:, 0]]
    r = e[pairs[:, 1]]
    logits = (l @ r.T) / temp
    lbl = jnp.arange(pairs.shape[0])
    a = jax.nn.log_softmax(logits, axis=-1)[lbl, lbl]
    b = jax.nn.log_softmax(logits.T, axis=-1)[lbl, lbl]
    return -0.5 * (jnp.mean(a) + jnp.mean(b))


def kernel(train_paris, flag, adj_matrix, r_index, r_val, rel_matrix,
           att_matrix, ent_matrix, high_adj, ill_ent, ent_semantic_emb,
           rel_semantic_emb, att_semantic_emb, ent_emb, rel_emb, att_emb,
           e_att, e_bias, r_att, r_bias, a_att, a_bias, ent_W1, ent_b1,
           ent_W2, ent_b2, rel_W1, rel_b1, rel_W2, rel_b2, att_W1, att_b1,
           att_W2, att_b2, g_al, g_ar):
    H3 = _avg3(ent_matrix, rel_matrix, att_matrix, ent_emb, rel_emb, att_emb)

    rel_ids = r_index[1]
    rtab = rel_emb / (jnp.linalg.norm(rel_emb, axis=-1, keepdims=True) + 1e-9)
    rn = rtab[rel_ids]

    kg = _encoders_fused(
        H3, rn, adj_matrix, r_val, high_adj,
        [e_att, r_att, a_att], [e_bias, r_bias, a_bias])

    def padw(w):
        return jnp.pad(w, ((0, 0), (0, _H - w.shape[1])))

    def padw2(w):
        return jnp.pad(w, ((0, _H - w.shape[0]), (0, _H - w.shape[1])))

    def padb(b):
        return jnp.pad(b, (0, _H - b.shape[0]))

    sem = _sem_mlp3(
        ent_semantic_emb, rel_semantic_emb, att_semantic_emb,
        [padw(ent_W1), padw(rel_W1), padw(att_W1)],
        [padw2(ent_W2), padw2(rel_W2), padw2(att_W2)],
        jnp.stack([padb(ent_b1), padb(rel_b1), padb(att_b1)]),
        jnp.stack([padb(ent_b2), padb(rel_b2), padb(att_b2)]),
    )

    fo_in = jnp.concatenate([kg, sem], axis=-1)
    fo = _diff_gat(fo_in, ent_matrix, g_al, g_ar)
    out = jnp.concatenate([kg, sem, fo], axis=-1)
    total = (_align_loss(kg, train_paris) + _align_loss(sem, train_paris)
             + _align_loss(out, train_paris))
    return total


# speedup vs baseline: 1.4966x; 1.0716x over previous
import functools

import jax
import jax.numpy as jnp
from jax import lax
from jax.experimental import pallas as pl
from jax.experimental.pallas import tpu as pltpu
from jax.experimental.pallas import tpu_sc as plsc

N = 10000
R = 2000
A = 1000
T = 160000
D = 250
DEPTH = 2


_RB = 400
_KB = 512
_H = 512


def _mlp3_body(x1, x2, x3, w11, w12, w13, w21, w22, w23,
               b1s, b2s, out, acc1, acc2, acc3):
    k = pl.program_id(1)
    nk = pl.num_programs(1)

    @pl.when(k == 0)
    def _init():
        acc1[...] = jnp.zeros_like(acc1)
        acc2[...] = jnp.zeros_like(acc2)
        acc3[...] = jnp.zeros_like(acc3)

    acc1[...] += jnp.dot(x1[...], w11[...], preferred_element_type=jnp.float32)
    acc2[...] += jnp.dot(x2[...], w12[...], preferred_element_type=jnp.float32)
    acc3[...] += jnp.dot(x3[...], w13[...], preferred_element_type=jnp.float32)

    @pl.when(k == nk - 1)
    def _epilogue():
        s = None
        for acc, w2, j in ((acc1, w21, 0), (acc2, w22, 1), (acc3, w23, 2)):
            h = jnp.maximum(acc[...] + b1s[j, :][None, :], 0.0)
            r = jnp.maximum(
                jnp.dot(h, w2[...], preferred_element_type=jnp.float32)
                + b2s[j, :][None, :], 0.0) + h
            s = r if s is None else s + r
        out[...] = s


def _sem_mlp3(x1, x2, x3, w1s, w2s, b1s, b2s):
    grid = (N // _RB, 4096 // _KB)
    xspec = pl.BlockSpec((_RB, _KB), lambda i, k: (i, k))
    w1spec = pl.BlockSpec((_KB, _H), lambda i, k: (k, 0))
    w2spec = pl.BlockSpec((_H, _H), lambda i, k: (0, 0))
    bspec = pl.BlockSpec((3, _H), lambda i, k: (0, 0))
    out = pl.pallas_call(
        _mlp3_body,
        grid=grid,
        in_specs=[xspec, xspec, xspec, w1spec, w1spec, w1spec,
                  w2spec, w2spec, w2spec, bspec, bspec],
        out_specs=pl.BlockSpec((_RB, _H), lambda i, k: (i, 0)),
        out_shape=jax.ShapeDtypeStruct((N, _H), jnp.float32),
        scratch_shapes=[pltpu.VMEM((_RB, _H), jnp.float32)] * 3,
        compiler_params=pltpu.CompilerParams(
            dimension_semantics=("parallel", "arbitrary")),
    )(x1, x2, x3, *w1s, *w2s, b1s, b2s)
    return out[:, :500]


_NPAD = 10240
_EC = 512
_CB = 64


def _sc_multiblock(tabb, src2d, dst2d, entries, nout):
    mesh = plsc.VectorSubcoreMesh(core_axis_name="c", subcore_axis_name="s")

    @functools.partial(
        pl.kernel,
        out_type=jax.ShapeDtypeStruct((nout * _NPAD, _CB), jnp.float32),
        mesh=mesh,
        scratch_types=[
            pltpu.VMEM((4, 128), jnp.int32),
            pltpu.VMEM((4, 128), jnp.int32),
            pltpu.VMEM((_EC, _CB), jnp.float32),
            pltpu.VMEM((160, _CB), jnp.float32),
            pltpu.VMEM_SHARED((_NPAD, _CB), jnp.float32),
            pltpu.SemaphoreType.DMA,
        ],
        compiler_params=pltpu.CompilerParams(use_tc_tiling_on_sc=False),
    )
    def k(tab_hbm, src_hbm, dst_hbm, g_hbm, idx_a, idx_d, rows,
          zbuf, acc, sem):
        cid = lax.axis_index("c")
        sid = lax.axis_index("s")

        def zrow(i, _):
            for j in range(_CB // 16):
                zbuf[i, pl.ds(j * 16, 16)] = jnp.zeros((16,), jnp.float32)
            return 0
        lax.fori_loop(0, 160, zrow, 0)

        def basis(i, _):
            rows[i, pl.ds(0, 16)] = jnp.where(
                lax.iota(jnp.int32, 16) == 0, 1.0, 0.0)
            for j in range(1, _CB // 16):
                rows[i, pl.ds(j * 16, 16)] = jnp.zeros((16,), jnp.float32)
            return 0

        def edge_pass(tab_base, erow_base, cpt, out_blk, gather):
            if not gather:
                lax.fori_loop(0, _EC, basis, 0)
            for r in range(4):
                pltpu.sync_copy(zbuf,
                                acc.at[pl.ds(sid * 640 + r * 160, 160), :])
            plsc.subcore_barrier()

            def per_chunk(kk, _):
                rbase = erow_base + (sid * cpt + kk) * (_EC // 128)
                pltpu.sync_copy(dst_hbm.at[pl.ds(rbase, 4), :], idx_d)
                if gather:
                    pltpu.sync_copy(src_hbm.at[pl.ds(rbase, 4), :], idx_a)
                    if tab_base:
                        def adj(i, _):
                            for j in range(4):
                                idx_a[j, pl.ds(i * 16, 16)] = (
                                    idx_a[j, pl.ds(i * 16, 16)] + tab_base)
                            return 0
                        lax.fori_loop(0, 8, adj, 0)
                    for j in range(4):
                        pltpu.async_copy(
                            tab_hbm.at[idx_a.at[j]],
                            rows.at[pl.ds(j * 128, 128), :], sem).wait()
                for j in range(4):
                    pltpu.sync_copy(rows.at[pl.ds(j * 128, 128), :],
                                    acc.at[idx_d.at[j]], add=True)
                return 0
            lax.fori_loop(0, cpt, per_chunk, 0)
            plsc.subcore_barrier()
            pltpu.sync_copy(
                acc.at[pl.ds(sid * 640, 640), :],
                g_hbm.at[pl.ds(out_blk * _NPAD + sid * 640, 640), :])
            plsc.subcore_barrier()

        for i, (tb, eb, cpt, ob, ga) in enumerate(entries):
            pl.when(cid == i % 2)(
                functools.partial(edge_pass, tb, eb, cpt, ob, ga))

    return k(tabb, src2d, dst2d)


_ECA = 128


def _sc_encoder_scores(Hrm, RTrm, hatt, ratt, src2d, src3d, rid2d, dst2d,
                       rval1d):
    TP = src2d.shape[0] * 128
    nrows = TP // 128
    cpt = TP // (32 * _ECA)
    mesh = plsc.VectorSubcoreMesh(core_axis_name="c", subcore_axis_name="s")

    @functools.partial(
        pl.kernel,
        out_type=(jax.ShapeDtypeStruct((TP * 16,), jnp.float32),
                  jax.ShapeDtypeStruct((TP * 16,), jnp.float32),
                  jax.ShapeDtypeStruct((2 * _NPAD, 64), jnp.float32)),
        mesh=mesh,
        scratch_types=[
            pltpu.VMEM((1, 128), jnp.int32),
            pltpu.VMEM((1, 128), jnp.int32),
            pltpu.VMEM((1, 128), jnp.int32),
            pltpu.VMEM((1, 128), jnp.int32),
            pltpu.VMEM((_ECA, 256), jnp.float32),
            pltpu.VMEM((_ECA, 256), jnp.float32),
            pltpu.VMEM((_ECA, 16), jnp.float32),
            pltpu.VMEM((_ECA, 16), jnp.float32),
            pltpu.VMEM((_ECA * 16,), jnp.float32),
            pltpu.VMEM((_ECA * 16,), jnp.float32),
            pltpu.VMEM((_ECA * 16,), jnp.float32),
            pltpu.VMEM((_ECA * 16,), jnp.float32),
            pltpu.VMEM((_ECA * 16,), jnp.float32),
            pltpu.VMEM((_ECA, 64), jnp.float32),
            pltpu.VMEM((_ECA,), jnp.float32),
            pltpu.VMEM_SHARED((_NPAD, 64), jnp.float32),
            pltpu.SemaphoreType.DMA,
        ],
        compiler_params=pltpu.CompilerParams(use_tc_tiling_on_sc=False,
                                             needs_layout_passes=False),
    )
    def k(h_hbm, rt_hbm, hatt_hbm, ratt_hbm, src_hbm, src3_hbm, rid_hbm,
          dst_hbm, rval_hbm, dout_hbm, eout_hbm, den_hbm, ixs, ixa, ixr, ixd,
          hs, rn, ha, ra, ha1, ra1, dd1, ee1, ab1, dr, rv, den, sem):
        cid = lax.axis_index("c")
        sid = lax.axis_index("s")
        wid = cid * 16 + sid

        def zdr(i, _):
            for q in range(4):
                dr[i, pl.ds(q * 16, 16)] = jnp.zeros((16,), jnp.float32)
            return 0
        lax.fori_loop(0, _ECA, zdr, 0)
        for r in range(5):
            pltpu.sync_copy(dr,
                            den.at[pl.ds(sid * 640 + r * 128, 128), :])
        plsc.subcore_barrier()

        iot = lax.iota(jnp.int32, 16)

        def per_chunk(kk, _):
            rbase = wid * cpt + kk
            pltpu.sync_copy(src_hbm.at[pl.ds(rbase, 1), :], ixs)
            pltpu.sync_copy(rid_hbm.at[pl.ds(rbase, 1), :], ixr)
            pltpu.sync_copy(dst_hbm.at[pl.ds(rbase, 1), :], ixd)
            pltpu.sync_copy(rval_hbm.at[pl.ds(rbase * 128, _ECA)], rv)
            pltpu.async_copy(rt_hbm.at[ixr.at[0]], rn, sem).wait()
            pltpu.async_copy(hatt_hbm.at[ixs.at[0]], ha, sem).wait()
            pltpu.async_copy(ratt_hbm.at[ixr.at[0]], ra, sem).wait()

            @pl.when(kk == 0)
            def _z1():
                def z1(e, _):
                    z = jnp.zeros((16,), jnp.float32)
                    dd1[pl.ds(e * 16, 16)] = z
                    ee1[pl.ds(e * 16, 16)] = z
                    return 0
                lax.fori_loop(0, _ECA, z1, 0)

            def flat(e, _):
                ha1[pl.ds(e * 16, 16)] = ha[e, :]
                ra1[pl.ds(e * 16, 16)] = ra[e, :]
                return 0
            lax.fori_loop(0, _ECA, flat, 0)

            for j in range(3):
                pltpu.sync_copy(
                    src3_hbm.at[pl.ds(j * nrows + rbase, 1), :], ixa)
                pltpu.async_copy(h_hbm.at[ixa.at[0]], hs, sem).wait()

                def dot(e, _):
                    acc = hs[e, pl.ds(0, 16)] * rn[e, pl.ds(0, 16)]
                    for q in range(1, 16):
                        acc = acc + (hs[e, pl.ds(q * 16, 16)]
                                     * rn[e, pl.ds(q * 16, 16)])
                    ab1[pl.ds(e * 16, 16)] = acc * 2.0
                    return 0
                lax.fori_loop(0, _ECA, dot, 0)

                def hred(g, _):
                    fb = g * 256 + iot * 16
                    dcol = plsc.load_gather(ab1, [fb])
                    for c in range(1, 16):
                        dcol = dcol + plsc.load_gather(ab1, [fb + c])
                    hacol = plsc.load_gather(ha1, [fb + j])
                    racol = plsc.load_gather(ra1, [fb + j])
                    sv = hacol - dcol * racol
                    sv = (jnp.where(sv > 0, sv, 0.01 * sv)
                          + rv[pl.ds(g * 16, 16)])
                    ecol = jnp.exp(sv)
                    plsc.store_scatter(dd1, [fb + j], dcol)
                    plsc.store_scatter(ee1, [fb + j], ecol)
                    return 0
                lax.fori_loop(0, _ECA // 16, hred, 0)

            def todr(e, _):
                dr[e, pl.ds(0, 16)] = ee1[pl.ds(e * 16, 16)]
                return 0
            lax.fori_loop(0, _ECA, todr, 0)

            base = rbase * 128 * 16
            pltpu.sync_copy(dd1, dout_hbm.at[pl.ds(base, _ECA * 16)])
            pltpu.sync_copy(ee1, eout_hbm.at[pl.ds(base, _ECA * 16)])
            pltpu.sync_copy(dr, den.at[ixd.at[0]], add=True)
            return 0
        lax.fori_loop(0, cpt, per_chunk, 0)
        plsc.subcore_barrier()
        pltpu.sync_copy(
            den.at[pl.ds(sid * 640, 640), :],
            den_hbm.at[pl.ds(cid * _NPAD + sid * 640, 640), :])

    return k(Hrm, RTrm, hatt, ratt, src2d, src3d, rid2d, dst2d, rval1d)


def _sc_encoder_agg(Hb, RTb, dtab, etab, dentab, srcb2d, ridb2d, dst2d,
                    dst1d):
    TP = dst2d.shape[0] * 128
    nrows = TP // 128
    cpt = TP // (16 * _EC)
    mesh = plsc.VectorSubcoreMesh(core_axis_name="c", subcore_axis_name="s")

    @functools.partial(
        pl.kernel,
        out_type=jax.ShapeDtypeStruct((12 * _NPAD, _CB), jnp.float32),
        mesh=mesh,
        scratch_types=[
            pltpu.VMEM((4, 128), jnp.int32),
            pltpu.VMEM((4, 128), jnp.int32),
            pltpu.VMEM((_EC,), jnp.int32),
            pltpu.VMEM((_EC, _CB), jnp.float32),
            pltpu.VMEM((_EC, _CB), jnp.float32),
            pltpu.VMEM((_EC * 16,), jnp.float32),
            pltpu.VMEM((_EC * 16,), jnp.float32),
            pltpu.VMEM((_NPAD,), jnp.float32),
            pltpu.VMEM((_NPAD,), jnp.float32),
            pltpu.VMEM((_NPAD,), jnp.float32),
            pltpu.VMEM((_EC,), jnp.float32),
            pltpu.VMEM((_EC,), jnp.float32),
            pltpu.SMEM((_EC,), jnp.float32),
            pltpu.SMEM((_EC,), jnp.float32),
            pltpu.VMEM((80, _CB), jnp.float32),
            pltpu.VMEM_SHARED((_NPAD, _CB), jnp.float32),
            pltpu.SemaphoreType.DMA,
        ],
        compiler_params=pltpu.CompilerParams(use_tc_tiling_on_sc=False,
                                             needs_layout_passes=False),
    )
    def k(h_hbm, rt_hbm, d_hbm, e_hbm, den_hbm, src_hbm, rid_hbm, dst_hbm,
          dst1_hbm, g_hbm, ixa, ixd, ixd1, hs, rn, dd, ee, dn0,
          dn1, dn2, ac, bc, asm, bsm, zbuf, acc, sem):
        cid = lax.axis_index("c")
        sid = lax.axis_index("s")

        def zrow(i, _):
            for q in range(_CB // 16):
                zbuf[i, pl.ds(q * 16, 16)] = jnp.zeros((16,), jnp.float32)
            return 0
        lax.fori_loop(0, 80, zrow, 0)
        iot = lax.iota(jnp.int32, 16)
        pltpu.sync_copy(den_hbm.at[pl.ds(0, _NPAD)], dn0)
        pltpu.sync_copy(den_hbm.at[pl.ds(_NPAD, _NPAD)], dn1)
        pltpu.sync_copy(den_hbm.at[pl.ds(2 * _NPAD, _NPAD)], dn2)

        def block_pass(bb):
            blk = 2 * bb + cid
            jrow = blk // 4
            rblk = blk % 4
            for r in range(8):
                pltpu.sync_copy(zbuf,
                                acc.at[pl.ds(sid * 640 + r * 80, 80), :])
            plsc.subcore_barrier()

            def per_chunk(kk, _):
                rbase = (sid * cpt + kk) * (_EC // 128)
                pltpu.sync_copy(dst_hbm.at[pl.ds(rbase, 4), :], ixd)
                pltpu.sync_copy(dst1_hbm.at[pl.ds(rbase * 128, _EC)], ixd1)
                ebase = rbase * 128 * 16
                pltpu.sync_copy(d_hbm.at[pl.ds(ebase, _EC * 16)], dd)
                pltpu.sync_copy(e_hbm.at[pl.ds(ebase, _EC * 16)], ee)
                pltpu.sync_copy(
                    src_hbm.at[pl.ds(blk * nrows + rbase, 4), :], ixa)
                for q in range(4):
                    pltpu.async_copy(h_hbm.at[ixa.at[q]],
                                     hs.at[pl.ds(q * 128, 128), :],
                                     sem).wait()
                pltpu.sync_copy(
                    rid_hbm.at[pl.ds(rblk * nrows + rbase, 4), :], ixa)
                for q in range(4):
                    pltpu.async_copy(rt_hbm.at[ixa.at[q]],
                                     rn.at[pl.ds(q * 128, 128), :],
                                     sem).wait()

                jr = iot * 0 + jrow

                def grp(g, _):
                    fb = g * 256 + iot * 16 + jrow
                    ecol = plsc.load_gather(ee, [fb])
                    dcol = plsc.load_gather(dd, [fb])
                    dstv = ixd1[pl.ds(g * 16, 16)]
                    da = plsc.load_gather(dn0, [dstv])
                    db = plsc.load_gather(dn1, [dstv])
                    dc = plsc.load_gather(dn2, [dstv])
                    dncol = jnp.where(jr == 0, da,
                                      jnp.where(jr == 1, db, dc))
                    al = ecol / (dncol + 1e-9)
                    ac[pl.ds(g * 16, 16)] = al
                    bc[pl.ds(g * 16, 16)] = -(al * dcol)
                    return 0
                lax.fori_loop(0, _EC // 16, grp, 0)
                pltpu.sync_copy(ac, asm)
                pltpu.sync_copy(bc, bsm)

                def scale(e, _):
                    a = 2.0
                    b = 3.0
                    for q in range(4):
                        hs[e, pl.ds(q * 16, 16)] = (
                            hs[e, pl.ds(q * 16, 16)] * a)
                        rn[e, pl.ds(q * 16, 16)] = (
                            rn[e, pl.ds(q * 16, 16)] * b)
                    return 0
                lax.fori_loop(0, _EC, scale, 0)

                for q in range(4):
                    pltpu.sync_copy(hs.at[pl.ds(q * 128, 128), :],
                                    acc.at[ixd.at[q]], add=True)
                    pltpu.sync_copy(rn.at[pl.ds(q * 128, 128), :],
                                    acc.at[ixd.at[q]], add=True)
                return 0
            lax.fori_loop(0, cpt, per_chunk, 0)
            plsc.subcore_barrier()
            pltpu.sync_copy(
                acc.at[pl.ds(sid * 640, 640), :],
                g_hbm.at[pl.ds(blk * _NPAD + sid * 640, 640), :])
            plsc.subcore_barrier()

        for bb in range(6):
            block_pass(bb)

    return k(Hb, RTb, dtab, etab, dentab, srcb2d, ridb2d, dst2d, dst1d)


def _pad_edges(src, dst, dump, multiple=16 * _EC):
    t = src.shape[0]
    tp = ((t + multiple - 1) // multiple) * multiple
    pad = tp - t
    src = jnp.concatenate([src, jnp.zeros((pad,), jnp.int32)])
    dst = jnp.concatenate([dst, jnp.full((pad,), dump, jnp.int32)])
    return src, dst


def _seg_softmax(v, seg, num):
    m = jax.ops.segment_max(v, seg, num_segments=num)
    m = jnp.where(jnp.isfinite(m), m, 0.0)
    e = jnp.exp(v - m[seg])
    s = jax.ops.segment_sum(e, seg, num_segments=num)
    return e / (s[seg] + 1e-9)


def _avg3(ent_mat, rel_mat, att_mat, ent_emb, rel_emb, att_emb):
    def padt(t):
        return jnp.pad(t, ((0, _NPAD - t.shape[0]), (0, 256 - t.shape[1])))

    tabp = jnp.stack([padt(ent_emb), padt(rel_emb), padt(att_emb)])
    tabb = tabp.reshape(3, _NPAD, 4, _CB).transpose(0, 2, 1, 3)
    tabb = tabb.reshape(12 * _NPAD, _CB)

    srcs, dsts, ebase, cpts = [], [], [], []
    rb = 0
    for mat in (ent_mat, rel_mat, att_mat):
        c, d = _pad_edges(mat[1], mat[0], N)
        srcs.append(c)
        dsts.append(d)
        ebase.append(rb)
        rb += c.shape[0] // 128
        cpts.append(c.shape[0] // (16 * _EC))
    src2d = jnp.concatenate(srcs).reshape(-1, 128)
    dst2d = jnp.concatenate(dsts).reshape(-1, 128)

    entries = [((j * 4 + q) * _NPAD, ebase[j], cpts[j], j * 4 + q, True)
               for j in range(3) for q in range(4)]
    entries += [(0, ebase[j], cpts[j], 12 + j, False) for j in range(3)]
    r = _sc_multiblock(tabb, src2d, dst2d, entries, 15)
    r = r.reshape(15, _NPAD, _CB)
    s = r[:12].reshape(3, 4, _NPAD, _CB).transpose(2, 0, 1, 3)
    s = s.reshape(_NPAD, 3, 256)[:N, :, :250]
    cnt = r[12:15, :N, 0]
    return s / (cnt.T[:, :, None] + 1e-9)


def _encoders_fused(H3, rtab, adj, rid, r_val, high_adj, atts, biases):
    src, dst = adj[0], adj[1]
    rn = rtab[rid]
    att_l = [jnp.stack([a[l] for a in atts]) for l in range(DEPTH)]
    bias_l = [jnp.stack([b[l] for b in biases]) for l in range(DEPTH)]
    outs = []
    for l in range(DEPTH):
        Hs = H3[src]
        dj = jnp.einsum('tjc,tc->tj', Hs, rn)
        msg = Hs - 2.0 * dj[..., None] * rn[:, None, :]
        sc = (jax.nn.leaky_relu(jnp.einsum('tjc,jc->tj', msg, att_l[l]))
              + r_val[:, None])
        e = jnp.exp(sc)
        sm = jax.ops.segment_sum(e, dst, num_segments=N)
        alpha = e / (sm[dst] + 1e-9)
        agg = jax.ops.segment_sum(
            (alpha[..., None] * msg).reshape(T, 750), dst, num_segments=N)
        H3 = jnp.tanh(agg.reshape(N, 3, 250) + bias_l[l][None])
        outs.append(H3)

    OUT = jnp.stack(outs, axis=2).reshape(N, 3 * DEPTH * 250)
    nbl = 24
    OUTb = jnp.pad(OUT, ((0, 0), (0, nbl * _CB - 1500)))
    OUTb = OUTb.reshape(N, nbl, _CB).transpose(1, 0, 2).reshape(nbl * N, _CB)
    hsrc, hdst = _pad_edges(high_adj[0], high_adj[1], N)
    cpt = hsrc.shape[0] // (16 * _EC)
    entries = [(b * N, 0, cpt, b, True) for b in range(nbl)]
    entries += [(0, 0, cpt, nbl, False)]
    r = _sc_multiblock(OUTb, hsrc.reshape(-1, 128), hdst.reshape(-1, 128),
                       entries, nbl + 1).reshape(nbl + 1, _NPAD, _CB)
    g = r[:nbl, :N].transpose(1, 0, 2).reshape(N, nbl * _CB)[:, :1500]
    c = r[nbl, :N, 0]
    return OUT + g / (c[:, None] + 1e-9)


def _diff_gat(x, adj, al, ar):
    src, dst = adj[0], adj[1]
    sl = x @ al
    sr = x @ ar
    score = jax.nn.leaky_relu(sl[src] + sr[dst])
    alpha = _seg_softmax(score, dst, N)
    return jnp.tanh(jax.ops.segment_sum(alpha[:, None] * x[src], dst,
                                        num_segments=N))


def _norm(x):
    return x / (jnp.linalg.norm(x, axis=-1, keepdims=True) + 1e-5)


def _align_loss(emb, pairs, temp=0.1):
    e = _norm(emb)
    l = e[pairs[:, 0]]
    r = e[pairs[:, 1]]
    logits = (l @ r.T) / temp
    lbl = jnp.arange(pairs.shape[0])
    a = jax.nn.log_softmax(logits, axis=-1)[lbl, lbl]
    b = jax.nn.log_softmax(logits.T, axis=-1)[lbl, lbl]
    return -0.5 * (jnp.mean(a) + jnp.mean(b))


def kernel(train_paris, flag, adj_matrix, r_index, r_val, rel_matrix,
           att_matrix, ent_matrix, high_adj, ill_ent, ent_semantic_emb,
           rel_semantic_emb, att_semantic_emb, ent_emb, rel_emb, att_emb,
           e_att, e_bias, r_att, r_bias, a_att, a_bias, ent_W1, ent_b1,
           ent_W2, ent_b2, rel_W1, rel_b1, rel_W2, rel_b2, att_W1, att_b1,
           att_W2, att_b2, g_al, g_ar):
    H3 = _avg3(ent_matrix, rel_matrix, att_matrix, ent_emb, rel_emb, att_emb)

    rtab = rel_emb / (jnp.linalg.norm(rel_emb, axis=-1, keepdims=True) + 1e-9)

    kg = _encoders_fused(
        H3, rtab, adj_matrix, r_index[1], r_val, high_adj,
        [e_att, r_att, a_att], [e_bias, r_bias, a_bias])

    def padw(w):
        return jnp.pad(w, ((0, 0), (0, _H - w.shape[1])))

    def padw2(w):
        return jnp.pad(w, ((0, _H - w.shape[0]), (0, _H - w.shape[1])))

    def padb(b):
        return jnp.pad(b, (0, _H - b.shape[0]))

    sem = _sem_mlp3(
        ent_semantic_emb, rel_semantic_emb, att_semantic_emb,
        [padw(ent_W1), padw(rel_W1), padw(att_W1)],
        [padw2(ent_W2), padw2(rel_W2), padw2(att_W2)],
        jnp.stack([padb(ent_b1), padb(rel_b1), padb(att_b1)]),
        jnp.stack([padb(ent_b2), padb(rel_b2), padb(att_b2)]),
    )

    fo_in = jnp.concatenate([kg, sem], axis=-1)
    fo = _diff_gat(fo_in, ent_matrix, g_al, g_ar)
    out = jnp.concatenate([kg, sem, fo], axis=-1)
    total = (_align_loss(kg, train_paris) + _align_loss(sem, train_paris)
             + _align_loss(out, train_paris))
    return total
